# baseline (speedup 1.0000x reference)
import jax
import jax.numpy as jnp
from jax.experimental import pallas as pl
from jax.experimental.pallas import tpu as pltpu

N = 64
B = 128
BB = 8


def _ring_agg(h):
    hm = jnp.concatenate([h[:, -1:, :], h[:, :-1, :]], axis=1)
    hp = jnp.concatenate([h[:, 1:, :], h[:, :1, :]], axis=1)
    return (hm + hp) * 0.5


def _dot(a, w):
    return jnp.dot(a.astype(jnp.bfloat16), w,
                   preferred_element_type=jnp.float32)


def _gnn_layer(h, W, b):
    F = h.shape[-1]
    agg = _ring_agg(h)
    h2 = h.reshape(BB * N, F)
    a2 = agg.reshape(BB * N, F)
    y = _dot(h2, W[:F]) + _dot(a2, W[F:]) + b
    return jnp.maximum(y, 0.0).reshape(BB, N, -1)


def _body(x_ref, Wg0, bg0, Wg1, bg1, Wg2, bg2, Wmu, bmu, Wls, bls,
          Wd0, bd0, Wd1, bd1, Wd2, bd2, Wom, bom, Wos, bos,
          epsz_ref, epso_ref, out_ref):
    h = x_ref[...]
    h = _gnn_layer(h, Wg0[...], bg0[...])
    h = _gnn_layer(h, Wg1[...], bg1[...])
    h = _gnn_layer(h, Wg2[...], bg2[...])
    h2 = h.reshape(BB * N, 384)
    mu = _dot(h2, Wmu[...]) + bmu[...]
    logvar = _dot(h2, Wls[...]) + bls[...]
    z2 = mu + jnp.exp(0.5 * logvar) * epsz_ref[...].reshape(BB * N, 384)
    z = z2.reshape(BB, N, 384)
    d = _gnn_layer(z, Wd0[...], bd0[...])
    d = _gnn_layer(d, Wd1[...], bd1[...])
    d = _gnn_layer(d, Wd2[...], bd2[...])
    d2 = d.reshape(BB * N, 768)
    out_mu = _dot(d2, Wom[...]) + bom[...]
    out_sig = jax.nn.softplus(_dot(d2, Wos[...]) + bos[...])
    out = jnp.exp(out_mu + out_sig * epso_ref[...].reshape(BB * N, 768))
    out_ref[...] = out.reshape(BB, N, 768)


def _w_spec(shape):
    return pl.BlockSpec(shape, lambda i: (0,) * len(shape))


def kernel(x, Wg0, bg0, Wg1, bg1, Wg2, bg2, Wmu, bmu, Wls, bls,
           Wd0, bd0, Wd1, bd1, Wd2, bd2, Wom, bom, Wos, bos,
           eps_z, eps_out):
    biases = [b.reshape(1, -1) for b in (bg0, bg1, bg2, bmu, bls, bd0, bd1, bd2, bom, bos)]
    bg0, bg1, bg2, bmu, bls, bd0, bd1, bd2, bom, bos = biases
    Wg0, Wg1, Wg2, Wmu, Wls, Wd0, Wd1, Wd2, Wom, Wos = [
        w.astype(jnp.bfloat16)
        for w in (Wg0, Wg1, Wg2, Wmu, Wls, Wd0, Wd1, Wd2, Wom, Wos)]
    weights = (Wg0, bg0, Wg1, bg1, Wg2, bg2, Wmu, bmu, Wls, bls,
               Wd0, bd0, Wd1, bd1, Wd2, bd2, Wom, bom, Wos, bos)
    grid = (B // BB,)
    batch_spec = lambda f: pl.BlockSpec((BB, N, f), lambda i: (i, 0, 0))
    in_specs = [batch_spec(768)]
    in_specs += [_w_spec(w.shape) for w in weights]
    in_specs += [batch_spec(384), batch_spec(768)]
    return pl.pallas_call(
        _body,
        grid=grid,
        in_specs=in_specs,
        out_specs=batch_spec(768),
        out_shape=jax.ShapeDtypeStruct((B, N, 768), jnp.float32),
        compiler_params=pltpu.CompilerParams(
            dimension_semantics=("arbitrary",),
            vmem_limit_bytes=100 * 1024 * 1024,
        ),
    )(x, *weights, eps_z, eps_out)


# speedup vs baseline: 1.0364x; 1.0364x over previous
import jax
import jax.numpy as jnp
from jax.experimental import pallas as pl
from jax.experimental.pallas import tpu as pltpu

N = 64
B = 128
BB = 16


def _ring_agg(h):
    hm = jnp.concatenate([h[:, -1:, :], h[:, :-1, :]], axis=1)
    hp = jnp.concatenate([h[:, 1:, :], h[:, :1, :]], axis=1)
    return (hm + hp) * 0.5


def _dot(a, w):
    return jnp.dot(a.astype(jnp.bfloat16), w,
                   preferred_element_type=jnp.float32)


def _gnn_layer(h, W, b):
    F = h.shape[-1]
    agg = _ring_agg(h)
    h2 = h.reshape(BB * N, F)
    a2 = agg.reshape(BB * N, F)
    y = _dot(h2, W[:F]) + _dot(a2, W[F:]) + b
    return jnp.maximum(y, 0.0).reshape(BB, N, -1)


def _body(x_ref, Wg0, bg0, Wg1, bg1, Wg2, bg2, Wmu, bmu, Wls, bls,
          Wd0, bd0, Wd1, bd1, Wd2, bd2, Wom, bom, Wos, bos,
          epsz_ref, epso_ref, out_ref):
    h = x_ref[...]
    h = _gnn_layer(h, Wg0[...], bg0[...])
    h = _gnn_layer(h, Wg1[...], bg1[...])
    h = _gnn_layer(h, Wg2[...], bg2[...])
    h2 = h.reshape(BB * N, 384)
    mu = _dot(h2, Wmu[...]) + bmu[...]
    logvar = _dot(h2, Wls[...]) + bls[...]
    z2 = mu + jnp.exp(0.5 * logvar) * epsz_ref[...].reshape(BB * N, 384)
    z = z2.reshape(BB, N, 384)
    d = _gnn_layer(z, Wd0[...], bd0[...])
    d = _gnn_layer(d, Wd1[...], bd1[...])
    d = _gnn_layer(d, Wd2[...], bd2[...])
    d2 = d.reshape(BB * N, 768)
    out_mu = _dot(d2, Wom[...]) + bom[...]
    out_sig = jax.nn.softplus(_dot(d2, Wos[...]) + bos[...])
    out = jnp.exp(out_mu + out_sig * epso_ref[...].reshape(BB * N, 768))
    out_ref[...] = out.reshape(BB, N, 768)


def _w_spec(shape):
    return pl.BlockSpec(shape, lambda i: (0,) * len(shape))


def kernel(x, Wg0, bg0, Wg1, bg1, Wg2, bg2, Wmu, bmu, Wls, bls,
           Wd0, bd0, Wd1, bd1, Wd2, bd2, Wom, bom, Wos, bos,
           eps_z, eps_out):
    biases = [b.reshape(1, -1) for b in (bg0, bg1, bg2, bmu, bls, bd0, bd1, bd2, bom, bos)]
    bg0, bg1, bg2, bmu, bls, bd0, bd1, bd2, bom, bos = biases
    Wg0, Wg1, Wg2, Wmu, Wls, Wd0, Wd1, Wd2, Wom, Wos = [
        w.astype(jnp.bfloat16)
        for w in (Wg0, Wg1, Wg2, Wmu, Wls, Wd0, Wd1, Wd2, Wom, Wos)]
    weights = (Wg0, bg0, Wg1, bg1, Wg2, bg2, Wmu, bmu, Wls, bls,
               Wd0, bd0, Wd1, bd1, Wd2, bd2, Wom, bom, Wos, bos)
    grid = (B // BB,)
    batch_spec = lambda f: pl.BlockSpec((BB, N, f), lambda i: (i, 0, 0))
    in_specs = [batch_spec(768)]
    in_specs += [_w_spec(w.shape) for w in weights]
    in_specs += [batch_spec(384), batch_spec(768)]
    return pl.pallas_call(
        _body,
        grid=grid,
        in_specs=in_specs,
        out_specs=batch_spec(768),
        out_shape=jax.ShapeDtypeStruct((B, N, 768), jnp.float32),
        compiler_params=pltpu.CompilerParams(
            dimension_semantics=("arbitrary",),
            vmem_limit_bytes=100 * 1024 * 1024,
        ),
    )(x, *weights, eps_z, eps_out)


# speedup vs baseline: 1.1345x; 1.0946x over previous
import jax
import jax.numpy as jnp
from jax.experimental import pallas as pl
from jax.experimental.pallas import tpu as pltpu

N = 64
B = 128
BB = 16


def _ring_agg(h):
    hm = jnp.concatenate([h[:, -1:, :], h[:, :-1, :]], axis=1)
    hp = jnp.concatenate([h[:, 1:, :], h[:, :1, :]], axis=1)
    return (hm + hp) * 0.5


def _dot(a, w):
    return jnp.dot(a.astype(jnp.bfloat16), w.astype(jnp.bfloat16),
                   preferred_element_type=jnp.float32)


def _gnn_layer(h, W, b):
    F = h.shape[-1]
    agg = _ring_agg(h)
    h2 = h.reshape(BB * N, F)
    a2 = agg.reshape(BB * N, F)
    y = _dot(h2, W[:F]) + _dot(a2, W[F:]) + b
    return jnp.maximum(y, 0.0).reshape(BB, N, -1)


def _body(x_ref, Wg0, bg0, Wg1, bg1, Wg2, bg2, Wmu, bmu, Wls, bls,
          Wd0, bd0, Wd1, bd1, Wd2, bd2, Wom, bom, Wos, bos,
          epsz_ref, epso_ref, out_ref):
    h = x_ref[...]
    h = _gnn_layer(h, Wg0[...], bg0[...])
    h = _gnn_layer(h, Wg1[...], bg1[...])
    h = _gnn_layer(h, Wg2[...], bg2[...])
    h2 = h.reshape(BB * N, 384)
    mu = _dot(h2, Wmu[...]) + bmu[...]
    logvar = _dot(h2, Wls[...]) + bls[...]
    z2 = mu + jnp.exp(0.5 * logvar) * epsz_ref[...].reshape(BB * N, 384)
    z = z2.reshape(BB, N, 384)
    d = _gnn_layer(z, Wd0[...], bd0[...])
    d = _gnn_layer(d, Wd1[...], bd1[...])
    d = _gnn_layer(d, Wd2[...], bd2[...])
    d2 = d.reshape(BB * N, 768)
    out_mu = _dot(d2, Wom[...]) + bom[...]
    out_sig = jax.nn.softplus(_dot(d2, Wos[...]) + bos[...])
    out = jnp.exp(out_mu + out_sig * epso_ref[...].reshape(BB * N, 768))
    out_ref[...] = out.reshape(BB, N, 768)


def _w_spec(shape):
    return pl.BlockSpec(shape, lambda i: (0,) * len(shape))


def kernel(x, Wg0, bg0, Wg1, bg1, Wg2, bg2, Wmu, bmu, Wls, bls,
           Wd0, bd0, Wd1, bd1, Wd2, bd2, Wom, bom, Wos, bos,
           eps_z, eps_out):
    biases = [b.reshape(1, -1) for b in (bg0, bg1, bg2, bmu, bls, bd0, bd1, bd2, bom, bos)]
    bg0, bg1, bg2, bmu, bls, bd0, bd1, bd2, bom, bos = biases
    weights = (Wg0, bg0, Wg1, bg1, Wg2, bg2, Wmu, bmu, Wls, bls,
               Wd0, bd0, Wd1, bd1, Wd2, bd2, Wom, bom, Wos, bos)
    grid = (B // BB,)
    batch_spec = lambda f: pl.BlockSpec((BB, N, f), lambda i: (i, 0, 0))
    in_specs = [batch_spec(768)]
    in_specs += [_w_spec(w.shape) for w in weights]
    in_specs += [batch_spec(384), batch_spec(768)]
    return pl.pallas_call(
        _body,
        grid=grid,
        in_specs=in_specs,
        out_specs=batch_spec(768),
        out_shape=jax.ShapeDtypeStruct((B, N, 768), jnp.float32),
        compiler_params=pltpu.CompilerParams(
            dimension_semantics=("arbitrary",),
            vmem_limit_bytes=100 * 1024 * 1024,
        ),
    )(x, *weights, eps_z, eps_out)
